# Initial kernel scaffold; baseline (speedup 1.0000x reference)
import jax
import jax.numpy as jnp
from jax.experimental import pallas as pl


def kernel(inputs):
    raise NotImplementedError("write your pallas kernel here")


# speedup vs baseline: 3.8784x; 3.8784x over previous
import jax
import jax.numpy as jnp
from jax.experimental import pallas as pl
from jax.experimental.pallas import tpu as pltpu

P = 0.2
B, C, D = 4, 4096, 2048
DROP_NUM = int(round(P * C))
SCALE = 1.0 / (1.0 - P)
CBLK = 512


def _compute_scale(rand):
    bits = jax.lax.bitcast_convert_type(rand, jnp.int32)

    lo = jnp.zeros((B, 1), jnp.int32)
    hi = jnp.full((B, 1), jnp.int32(0x3F800000))

    def body_val(_, carry):
        lo, hi = carry
        mid = lo + (hi - lo) // 2
        cnt = jnp.sum((bits <= mid).astype(jnp.int32), axis=1, keepdims=True)
        ge = cnt >= DROP_NUM
        return jnp.where(ge, lo, mid + 1), jnp.where(ge, mid, hi)

    lo, hi = jax.lax.fori_loop(0, 31, body_val, (lo, hi))
    tbits = lo

    less = bits < tbits
    eq = bits == tbits
    n_less = jnp.sum(less.astype(jnp.int32), axis=1, keepdims=True)
    need = DROP_NUM - n_less

    idx = jax.lax.broadcasted_iota(jnp.int32, (B, C), 1)
    ilo = jnp.full((B, 1), -1, jnp.int32)
    ihi = jnp.full((B, 1), C - 1, jnp.int32)

    def body_idx(_, carry):
        ilo, ihi = carry
        mid = ilo + (ihi - ilo + 1) // 2
        cnt = jnp.sum((eq & (idx <= mid)).astype(jnp.int32), axis=1,
                      keepdims=True)
        ge = cnt >= need
        return jnp.where(ge, ilo, mid), jnp.where(ge, mid, ihi)

    ilo, ihi = jax.lax.fori_loop(0, 13, body_idx, (ilo, ihi))
    itop = ihi

    drop = less | (eq & (idx <= itop))
    return jnp.where(drop, 0.0, SCALE).astype(jnp.float32)


def _fused_kernel(rand_ref, x_ref, o_ref, scale_ref):
    b = pl.program_id(0)
    c = pl.program_id(1)

    @pl.when((b == 0) & (c == 0))
    def _():
        scale_ref[...] = _compute_scale(rand_ref[...])

    s = scale_ref[b, pl.ds(c * CBLK, CBLK)]
    o_ref[0] = x_ref[0] * s[:, None]


@jax.jit
def kernel(inputs):
    rand = jax.random.uniform(jax.random.key(42), (B, C), dtype=jnp.float32)

    out = pl.pallas_call(
        _fused_kernel,
        grid=(B, C // CBLK),
        in_specs=[
            pl.BlockSpec((B, C), lambda b, c: (0, 0)),
            pl.BlockSpec((1, CBLK, D), lambda b, c: (b, c, 0)),
        ],
        out_specs=pl.BlockSpec((1, CBLK, D), lambda b, c: (b, c, 0)),
        out_shape=jax.ShapeDtypeStruct((B, C, D), jnp.float32),
        scratch_shapes=[pltpu.VMEM((B, C), jnp.float32)],
    )(rand, inputs)
    return out
